# Initial kernel scaffold; baseline (speedup 1.0000x reference)
import jax
import jax.numpy as jnp
from jax.experimental import pallas as pl


def kernel(norm, pos, params, edge_index, batch):
    raise NotImplementedError("write your pallas kernel here")


# speedup vs baseline: 1.4596x; 1.4596x over previous
import functools

import jax
import jax.numpy as jnp
from jax import lax
from jax.experimental import pallas as pl
from jax.experimental.pallas import tpu as pltpu
from jax.experimental.pallas import tpu_sc as plsc

B = 16
NPG = 625
NP = 640
EPG = 10000
HID = 32
K1, K2, K3 = 563, 507, 457
ROWS = 160
CHUNK = 2000
NLANE = 16


def _adj_body(src_hbm, dst_hbm, out_hbm, src_v, dst_v, acc_v):
    wid = lax.axis_index("s") * 2 + lax.axis_index("c")
    iota = lax.iota(jnp.int32, NLANE)
    ones = jnp.ones((NLANE,), jnp.float32)
    for half in range(2):
        t = wid + half * 32
        g = t // 4
        q = t % 4
        row0 = q * ROWS

        def zbody(i, carry):
            acc_v[pl.ds(i * NLANE, NLANE)] = jnp.zeros((NLANE,), jnp.float32)
            return carry

        lax.fori_loop(0, (ROWS * NP) // NLANE, zbody, 0)

        for ch in range(EPG // CHUNK):
            off = pl.multiple_of(g * EPG + ch * CHUNK, CHUNK)
            pltpu.sync_copy(src_hbm.at[pl.ds(off, CHUNK)], src_v)
            pltpu.sync_copy(dst_hbm.at[pl.ds(off, CHUNK)], dst_v)

            def ebody(i, carry):
                sv = src_v[pl.ds(i * NLANE, NLANE)]
                dv = dst_v[pl.ds(i * NLANE, NLANE)]
                c = sv - g * NPG
                r = dv - g * NPG - row0
                ok = (r >= 0) & (r < ROWS)
                key = jnp.clip(r * NP + c, 0, ROWS * NP - 1)
                for l in range(NLANE):
                    plsc.addupdate_scatter(
                        acc_v, [key], ones, mask=ok & (iota == l))
                return carry

            lax.fori_loop(0, CHUNK // NLANE, ebody, 0)

        dst_off = pl.multiple_of((g * NP + row0) * NP, ROWS * NP)
        pltpu.sync_copy(acc_v, out_hbm.at[pl.ds(dst_off, ROWS * NP)])


def _adj_call(src, dst):
    mesh = plsc.VectorSubcoreMesh(core_axis_name="c", subcore_axis_name="s")
    kern = functools.partial(
        pl.kernel,
        mesh=mesh,
        compiler_params=pltpu.CompilerParams(needs_layout_passes=False),
        out_type=jax.ShapeDtypeStruct((B * NP * NP,), jnp.float32),
        scratch_types=[
            pltpu.VMEM((CHUNK,), jnp.int32),
            pltpu.VMEM((CHUNK,), jnp.int32),
            pltpu.VMEM((ROWS * NP,), jnp.float32),
        ],
    )(_adj_body)
    return kern(src, dst).reshape(B, NP, NP)


def _unet_body(A_ref, x0_ref, W0_ref, Ws_ref, bs_ref, pp_ref,
               pooled_ref, argm_ref,
               A1s, A2s, P1s, P2s, xs0, xs1, xs2):
    f32 = jnp.float32
    rio = lax.broadcasted_iota(jnp.int32, (NP, NP), 0)
    cio = lax.broadcasted_iota(jnp.int32, (NP, NP), 1)
    eyef = (rio == cio).astype(f32)
    rcol = lax.broadcasted_iota(jnp.int32, (NP, 1), 0)
    rcolf = rcol.astype(f32)

    def tcol(v):
        return lax.dot_general(v, eyef, (((0,), (0,)), ((), ())),
                               preferred_element_type=f32,
                               precision=lax.Precision.HIGHEST)

    def gcn(A, x, W, b2):
        Ah = A + eyef
        deg = jnp.sum(Ah, axis=1, keepdims=True)
        dis = jnp.where(deg > 0, 1.0 / jnp.sqrt(deg), 0.0)
        An = Ah * dis * tcol(dis)
        xW = jnp.dot(x, W, preferred_element_type=f32)
        return jnp.dot(An, xW, preferred_element_type=f32) + b2

    def augment(A):
        Ai = A + eyef
        A2 = jnp.dot(Ai, Ai, preferred_element_type=f32)
        return A2 * (1.0 - eyef)

    def pool(A, x, p2, n_valid, k):
        nrm = jnp.sqrt(jnp.sum(p2 * p2))
        raw = jnp.sum(x * p2, axis=1, keepdims=True)
        s = jnp.tanh(raw / nrm)
        s = jnp.where(rcol < n_valid, s, -2.0)
        sT = tcol(s)
        gt = (sT > s).astype(f32)
        tie = ((sT == s) & (cio < rio)).astype(f32)
        rank = jnp.sum(gt + tie, axis=1, keepdims=True)
        P = ((tcol(rank) == rcolf) & (rcol < k)).astype(f32)
        hp = lax.Precision.HIGHEST
        vals = jnp.dot(P, s, preferred_element_type=f32, precision=hp)
        xn = jnp.dot(P, x, preferred_element_type=f32, precision=hp) * vals
        PA = jnp.dot(P, A, preferred_element_type=f32, precision=hp)
        An = lax.dot_general(PA, P, (((1,), (1,)), ((), ())),
                             preferred_element_type=f32, precision=hp)
        return An, xn, P

    def unpool(P, x):
        return lax.dot_general(P, x, (((0,), (0,)), ((), ())),
                               preferred_element_type=f32,
                               precision=lax.Precision.HIGHEST)

    A0 = A_ref[0]
    x0 = x0_ref[0]
    W0 = W0_ref[...]

    def Wb(i):
        return Ws_ref[i - 1], bs_ref[i:i + 1, :]

    x = jax.nn.relu(gcn(A0, x0, W0, bs_ref[0:1, :]))
    xs0[...] = x

    Aaug = augment(A0)
    A, x, P1 = pool(Aaug, x, pp_ref[0:1, :], NPG, K1)
    P1s[...] = P1
    A1s[...] = A
    W, b2 = Wb(1)
    x = jax.nn.relu(gcn(A, x, W, b2))
    xs1[...] = x

    Aaug = augment(A)
    A, x, P2 = pool(Aaug, x, pp_ref[1:2, :], K1, K2)
    P2s[...] = P2
    A2s[...] = A
    W, b2 = Wb(2)
    x = jax.nn.relu(gcn(A, x, W, b2))
    xs2[...] = x

    Aaug = augment(A)
    A, x, P3 = pool(Aaug, x, pp_ref[2:3, :], K2, K3)
    W, b2 = Wb(3)
    x = jax.nn.relu(gcn(A, x, W, b2))

    x = xs2[...] + unpool(P3, x)
    W, b2 = Wb(4)
    x = jax.nn.relu(gcn(A2s[...], x, W, b2))

    x = xs1[...] + unpool(P2s[...], x)
    W, b2 = Wb(5)
    x = jax.nn.relu(gcn(A1s[...], x, W, b2))

    x = xs0[...] + unpool(P1s[...], x)
    W, b2 = Wb(6)
    x = gcn(A0, x, W, b2)

    rio32 = lax.broadcasted_iota(jnp.int32, (NP, HID), 0)
    xm = jnp.where(rio32 < NPG, x, -3.4e38)
    mx = jnp.max(xm, axis=0, keepdims=True)
    cand = jnp.where(xm == mx, rio32, NP)
    pooled_ref[0] = mx
    argm_ref[0] = jnp.min(cand, axis=0, keepdims=True)


def _unet_call(A, x0, W0, Ws, bs, pp, interpret=False):
    grid = (B,)
    return pl.pallas_call(
        _unet_body,
        grid=grid,
        in_specs=[
            pl.BlockSpec((1, NP, NP), lambda b: (b, 0, 0)),
            pl.BlockSpec((1, NP, 8), lambda b: (b, 0, 0)),
            pl.BlockSpec((8, HID), lambda b: (0, 0)),
            pl.BlockSpec((6, HID, HID), lambda b: (0, 0, 0)),
            pl.BlockSpec((7, HID), lambda b: (0, 0)),
            pl.BlockSpec((3, HID), lambda b: (0, 0)),
        ],
        out_specs=[
            pl.BlockSpec((1, 1, HID), lambda b: (b, 0, 0)),
            pl.BlockSpec((1, 1, HID), lambda b: (b, 0, 0)),
        ],
        out_shape=[
            jax.ShapeDtypeStruct((B, 1, HID), jnp.float32),
            jax.ShapeDtypeStruct((B, 1, HID), jnp.int32),
        ],
        scratch_shapes=[
            pltpu.VMEM((NP, NP), jnp.float32),
            pltpu.VMEM((NP, NP), jnp.float32),
            pltpu.VMEM((NP, NP), jnp.float32),
            pltpu.VMEM((NP, NP), jnp.float32),
            pltpu.VMEM((NP, HID), jnp.float32),
            pltpu.VMEM((NP, HID), jnp.float32),
            pltpu.VMEM((NP, HID), jnp.float32),
        ],
        interpret=interpret,
    )(A, x0, W0, Ws, bs, pp)


def _head_body(h_ref, w0, b0, w1, b1, w2, b2, w3, b3, g0, g1, g2,
               bb0, bb1, bb2, out_ref):
    f32 = jnp.float32
    bn = 1.0 / jnp.sqrt(jnp.float32(1.0 + 1e-5))
    h = h_ref[...]
    for w, b, g, bb in ((w0, b0, g0, bb0), (w1, b1, g1, bb1),
                        (w2, b2, g2, bb2)):
        h = jax.nn.relu(jnp.dot(h, w[...], preferred_element_type=f32)
                        + b[...])
        h = h * bn * g[...] + bb[...]
    logits = jnp.dot(h, w3[...], preferred_element_type=f32) + b3[...]
    m = jnp.max(logits, axis=1, keepdims=True)
    shifted = logits - m
    lse = jnp.log(jnp.sum(jnp.exp(shifted), axis=1, keepdims=True))
    out_ref[...] = shifted - lse


def _head_call(pooled, hW, hb, bg, bb, interpret=False):
    args = [pooled,
            hW[0], hb[0][None, :], hW[1], hb[1][None, :],
            hW[2], hb[2][None, :], hW[3], hb[3][None, :],
            bg[0][None, :], bg[1][None, :], bg[2][None, :],
            bb[0][None, :], bb[1][None, :], bb[2][None, :]]
    return pl.pallas_call(
        _head_body,
        out_shape=jax.ShapeDtypeStruct((B, HID), jnp.float32),
        interpret=interpret,
    )(*args)


def kernel(norm, pos, params, edge_index, batch):
    x0 = jnp.concatenate([norm, pos], axis=1).reshape(B, NPG, 6)
    x0 = jnp.pad(x0, ((0, 0), (0, NP - NPG), (0, 2)))
    W0 = jnp.pad(params["gcn_W"][0], ((0, 2), (0, 0)))
    Ws = jnp.stack(params["gcn_W"][1:7])
    bs = jnp.stack(params["gcn_b"])
    pp = jnp.stack(params["pool_p"])

    A = _adj_call(edge_index[0], edge_index[1])
    pooled, argm = _unet_call(A, x0, W0, Ws, bs, pp)
    logp = _head_call(pooled.reshape(B, HID), params["head_W"],
                      params["head_b"], params["bn_g"], params["bn_b"])
    return logp, argm.reshape(B, HID)


# speedup vs baseline: 1.7918x; 1.2275x over previous
import functools

import jax
import jax.numpy as jnp
from jax import lax
from jax.experimental import pallas as pl
from jax.experimental.pallas import tpu as pltpu
from jax.experimental.pallas import tpu_sc as plsc

B = 16
NPG = 625
NP = 640
EPG = 10000
HID = 32
K1, K2, K3 = 563, 507, 457
N0, N1, N2 = 640, 576, 512
ROWS = 160
CHUNK = 2000
NLANE = 16


def _adj_body(src_hbm, dst_hbm, out_hbm, src_v, dst_v, acc_v):
    wid = lax.axis_index("s") * 2 + lax.axis_index("c")
    iota = lax.iota(jnp.int32, NLANE)
    ones = jnp.ones((NLANE,), jnp.float32)
    for half in range(2):
        t = wid + half * 32
        g = t // 4
        q = t % 4
        row0 = q * ROWS

        def zbody(i, carry):
            acc_v[pl.ds(i * NLANE, NLANE)] = jnp.zeros((NLANE,), jnp.float32)
            return carry

        lax.fori_loop(0, (ROWS * NP) // NLANE, zbody, 0)

        for ch in range(EPG // CHUNK):
            off = pl.multiple_of(g * EPG + ch * CHUNK, CHUNK)
            pltpu.sync_copy(src_hbm.at[pl.ds(off, CHUNK)], src_v)
            pltpu.sync_copy(dst_hbm.at[pl.ds(off, CHUNK)], dst_v)

            def ebody(i, carry):
                sv = src_v[pl.ds(i * NLANE, NLANE)]
                dv = dst_v[pl.ds(i * NLANE, NLANE)]
                c = sv - g * NPG
                r = dv - g * NPG - row0
                ok = (r >= 0) & (r < ROWS)
                key = jnp.clip(r * NP + c, 0, ROWS * NP - 1)
                for l in range(NLANE):
                    plsc.addupdate_scatter(
                        acc_v, [key], ones, mask=ok & (iota == l))
                return carry

            lax.fori_loop(0, CHUNK // NLANE, ebody, 0)

        dst_off = pl.multiple_of((g * NP + row0) * NP, ROWS * NP)
        pltpu.sync_copy(acc_v, out_hbm.at[pl.ds(dst_off, ROWS * NP)])


def _adj_call(src, dst):
    mesh = plsc.VectorSubcoreMesh(core_axis_name="c", subcore_axis_name="s")
    kern = functools.partial(
        pl.kernel,
        mesh=mesh,
        compiler_params=pltpu.CompilerParams(needs_layout_passes=False),
        out_type=jax.ShapeDtypeStruct((B * NP * NP,), jnp.float32),
        scratch_types=[
            pltpu.VMEM((CHUNK,), jnp.int32),
            pltpu.VMEM((CHUNK,), jnp.int32),
            pltpu.VMEM((ROWS * NP,), jnp.float32),
        ],
    )(_adj_body)
    return kern(src, dst).reshape(B, NP, NP)


def _unet_body(A_ref, x0_ref, W0_ref, Ws_ref, bs_ref, pp_ref,
               pooled_ref, argm_ref,
               A1s, A2s, P1s, P2s, xs0, xs1, xs2):
    f32 = jnp.float32
    hp = lax.Precision.HIGHEST

    def eye(n):
        r = lax.broadcasted_iota(jnp.int32, (n, n), 0)
        c = lax.broadcasted_iota(jnp.int32, (n, n), 1)
        return (r == c).astype(f32)

    eyes = {n: eye(n) for n in (N0, N1, N2)}

    def tcol(v):
        n = v.shape[0]
        return lax.dot_general(v, eyes[n], (((0,), (0,)), ((), ())),
                               preferred_element_type=f32, precision=hp)

    def gcn(A, x, W, b2):
        n = A.shape[0]
        Ah = A + eyes[n]
        deg = jnp.sum(Ah, axis=1, keepdims=True)
        dis = jnp.where(deg > 0, 1.0 / jnp.sqrt(deg), 0.0)
        An = Ah * dis * tcol(dis)
        xW = jnp.dot(x, W, preferred_element_type=f32)
        return jnp.dot(An, xW, preferred_element_type=f32) + b2

    def augment(A):
        n = A.shape[0]
        Ai = A + eyes[n]
        A2 = jnp.dot(Ai, Ai, preferred_element_type=f32)
        return A2 * (1.0 - eyes[n])

    def pool(A, x, p2, n_valid, k, m):
        n = A.shape[0]
        rcol = lax.broadcasted_iota(jnp.int32, (n, 1), 0)
        nrm = jnp.sqrt(jnp.sum(p2 * p2))
        raw = jnp.sum(x * p2, axis=1, keepdims=True)
        s = jnp.tanh(raw / nrm)
        s = jnp.where(rcol < n_valid, s, -2.0)
        sT = tcol(s)
        cio = lax.broadcasted_iota(jnp.int32, (n, n), 1)
        rio = lax.broadcasted_iota(jnp.int32, (n, n), 0)
        gt = (sT > s).astype(f32)
        tie = ((sT == s) & (cio < rio)).astype(f32)
        rank = jnp.sum(gt + tie, axis=1, keepdims=True)
        rio_m = lax.broadcasted_iota(jnp.int32, (m, 1), 0)
        P = ((tcol(rank) == rio_m.astype(f32)) & (rio_m < k)).astype(f32)
        vals = jnp.dot(P, s, preferred_element_type=f32, precision=hp)
        xn = jnp.dot(P, x, preferred_element_type=f32, precision=hp) * vals
        PA = jnp.dot(P, A, preferred_element_type=f32, precision=hp)
        An = lax.dot_general(PA, P, (((1,), (1,)), ((), ())),
                             preferred_element_type=f32, precision=hp)
        return An, xn, P

    def unpool(P, x):
        return lax.dot_general(P, x, (((0,), (0,)), ((), ())),
                               preferred_element_type=f32, precision=hp)

    A0 = A_ref[0]
    x0 = x0_ref[0]
    W0 = W0_ref[...]

    def Wb(i):
        return Ws_ref[i - 1], bs_ref[i:i + 1, :]

    x = jax.nn.relu(gcn(A0, x0, W0, bs_ref[0:1, :]))
    xs0[...] = x

    Aaug = augment(A0)
    A, x, P1 = pool(Aaug, x, pp_ref[0:1, :], NPG, K1, N1)
    P1s[...] = P1
    A1s[...] = A
    W, b2 = Wb(1)
    x = jax.nn.relu(gcn(A, x, W, b2))
    xs1[...] = x

    Aaug = augment(A)
    A, x, P2 = pool(Aaug, x, pp_ref[1:2, :], K1, K2, N2)
    P2s[...] = P2
    A2s[...] = A
    W, b2 = Wb(2)
    x = jax.nn.relu(gcn(A, x, W, b2))
    xs2[...] = x

    Aaug = augment(A)
    A, x, P3 = pool(Aaug, x, pp_ref[2:3, :], K2, K3, N2)
    W, b2 = Wb(3)
    x = jax.nn.relu(gcn(A, x, W, b2))

    x = xs2[...] + unpool(P3, x)
    W, b2 = Wb(4)
    x = jax.nn.relu(gcn(A2s[...], x, W, b2))

    x = xs1[...] + unpool(P2s[...], x)
    W, b2 = Wb(5)
    x = jax.nn.relu(gcn(A1s[...], x, W, b2))

    x = xs0[...] + unpool(P1s[...], x)
    W, b2 = Wb(6)
    x = gcn(A0, x, W, b2)

    rio32 = lax.broadcasted_iota(jnp.int32, (N0, HID), 0)
    xm = jnp.where(rio32 < NPG, x, -3.4e38)
    mx = jnp.max(xm, axis=0, keepdims=True)
    cand = jnp.where(xm == mx, rio32, N0)
    pooled_ref[0] = mx
    argm_ref[0] = jnp.min(cand, axis=0, keepdims=True)


def _unet_call(A, x0, W0, Ws, bs, pp, interpret=False):
    grid = (B,)
    return pl.pallas_call(
        _unet_body,
        grid=grid,
        in_specs=[
            pl.BlockSpec((1, NP, NP), lambda b: (b, 0, 0)),
            pl.BlockSpec((1, NP, 8), lambda b: (b, 0, 0)),
            pl.BlockSpec((8, HID), lambda b: (0, 0)),
            pl.BlockSpec((6, HID, HID), lambda b: (0, 0, 0)),
            pl.BlockSpec((7, HID), lambda b: (0, 0)),
            pl.BlockSpec((3, HID), lambda b: (0, 0)),
        ],
        out_specs=[
            pl.BlockSpec((1, 1, HID), lambda b: (b, 0, 0)),
            pl.BlockSpec((1, 1, HID), lambda b: (b, 0, 0)),
        ],
        out_shape=[
            jax.ShapeDtypeStruct((B, 1, HID), jnp.float32),
            jax.ShapeDtypeStruct((B, 1, HID), jnp.int32),
        ],
        scratch_shapes=[
            pltpu.VMEM((N1, N1), jnp.float32),
            pltpu.VMEM((N2, N2), jnp.float32),
            pltpu.VMEM((N1, N0), jnp.float32),
            pltpu.VMEM((N2, N1), jnp.float32),
            pltpu.VMEM((N0, HID), jnp.float32),
            pltpu.VMEM((N1, HID), jnp.float32),
            pltpu.VMEM((N2, HID), jnp.float32),
        ],
        interpret=interpret,
    )(A, x0, W0, Ws, bs, pp)


def _head_body(h_ref, w0, b0, w1, b1, w2, b2, w3, b3, g0, g1, g2,
               bb0, bb1, bb2, out_ref):
    f32 = jnp.float32
    bn = 1.0 / jnp.sqrt(jnp.float32(1.0 + 1e-5))
    h = h_ref[...]
    for w, b, g, bb in ((w0, b0, g0, bb0), (w1, b1, g1, bb1),
                        (w2, b2, g2, bb2)):
        h = jax.nn.relu(jnp.dot(h, w[...], preferred_element_type=f32)
                        + b[...])
        h = h * bn * g[...] + bb[...]
    logits = jnp.dot(h, w3[...], preferred_element_type=f32) + b3[...]
    m = jnp.max(logits, axis=1, keepdims=True)
    shifted = logits - m
    lse = jnp.log(jnp.sum(jnp.exp(shifted), axis=1, keepdims=True))
    out_ref[...] = shifted - lse


def _head_call(pooled, hW, hb, bg, bb, interpret=False):
    args = [pooled,
            hW[0], hb[0][None, :], hW[1], hb[1][None, :],
            hW[2], hb[2][None, :], hW[3], hb[3][None, :],
            bg[0][None, :], bg[1][None, :], bg[2][None, :],
            bb[0][None, :], bb[1][None, :], bb[2][None, :]]
    return pl.pallas_call(
        _head_body,
        out_shape=jax.ShapeDtypeStruct((B, HID), jnp.float32),
        interpret=interpret,
    )(*args)


def kernel(norm, pos, params, edge_index, batch):
    x0 = jnp.concatenate([norm, pos], axis=1).reshape(B, NPG, 6)
    x0 = jnp.pad(x0, ((0, 0), (0, NP - NPG), (0, 2)))
    W0 = jnp.pad(params["gcn_W"][0], ((0, 2), (0, 0)))
    Ws = jnp.stack(params["gcn_W"][1:7])
    bs = jnp.stack(params["gcn_b"])
    pp = jnp.stack(params["pool_p"])

    A = _adj_call(edge_index[0], edge_index[1])
    pooled, argm = _unet_call(A, x0, W0, Ws, bs, pp)
    logp = _head_call(pooled.reshape(B, HID), params["head_W"],
                      params["head_b"], params["bn_g"], params["bn_b"])
    return logp, argm.reshape(B, HID)
